# baseline (speedup 1.0000x reference)
import functools

import jax
import jax.numpy as jnp
from jax import lax
from jax.experimental import pallas as pl
from jax.experimental.pallas import tpu as pltpu
from jax.experimental.pallas import tpu_sc as plsc

_N = 9216
_B = 16
_S = 576
_D = 64
_K = 1024
_COMMIT = 0.25

_NW = 32
_RPW = _N // _NW
_CH = 48
_NCH = _RPW // _CH
_L = 16


def _tc_body(xt_ref, wt_ref, qt_ref, loss_ref, perp_ref, idx_ref,
             counts_ref, sse_ref, w2_ref, iota_ref):
    i = pl.program_id(0)
    xt = xt_ref[...].reshape(_D, _S)
    wt = wt_ref[...]

    @pl.when(i == 0)
    def _():
        w2_ref[...] = jnp.sum(wt * wt, axis=0, keepdims=True)
        iota_ref[...] = lax.broadcasted_iota(
            jnp.int32, (_S, _K), 1).astype(jnp.float32)

    x2 = jnp.sum(xt * xt, axis=0, keepdims=True)
    x2c = x2.reshape(_S, 1)
    m2 = lax.dot_general(xt * (-2.0), wt, (((0,), (0,)), ((), ())),
                         preferred_element_type=jnp.float32)
    dist = (x2c + w2_ref[...]) + m2

    dmin = jnp.min(dist, axis=1, keepdims=True)
    iota = iota_ref[...]
    idx = jnp.min(jnp.where(dist == dmin, iota, float(_K)), axis=1,
                  keepdims=True)

    idx_ref[...] = idx.reshape(1, 1, _S).astype(jnp.int32)

    onehot = (iota == idx).astype(jnp.float32)
    qt = lax.dot_general(wt, onehot, (((1,), (1,)), ((), ())),
                         preferred_element_type=jnp.float32)
    qt_ref[...] = (xt + (qt - xt)).reshape(1, _D, _S)

    part = jnp.sum((qt - xt) * (qt - xt))
    csum = jnp.sum(onehot, axis=0, keepdims=True)

    @pl.when(i == 0)
    def _():
        sse_ref[0] = part
        counts_ref[...] = csum

    @pl.when(i > 0)
    def _():
        sse_ref[0] += part
        counts_ref[...] += csum

    @pl.when(i == _B - 1)
    def _():
        mse = sse_ref[0] / float(_N * _D)
        loss_ref[0, 0] = mse + _COMMIT * mse
        avg = counts_ref[...] / float(_N)
        ent = jnp.sum(avg * jnp.log(avg + 1e-10))
        perp_ref[0, 0] = jnp.exp(-ent)


_sc_mesh = plsc.VectorSubcoreMesh(core_axis_name="c", subcore_axis_name="s")


@functools.partial(
    pl.kernel,
    out_type=jax.ShapeDtypeStruct((_N, _K), jnp.float32),
    mesh=_sc_mesh,
    compiler_params=pltpu.CompilerParams(needs_layout_passes=False),
    scratch_types=[
        pltpu.VMEM((_RPW,), jnp.int32),
        pltpu.VMEM((_CH, _K), jnp.float32),
    ],
)
def _sc_onehot(idx_hbm, enc_hbm, idx_v, buf):
    wid = lax.axis_index("s") * 2 + lax.axis_index("c")
    base = wid * _RPW
    pltpu.sync_copy(idx_hbm.at[pl.ds(base, _RPW)], idx_v)

    zeros16 = jnp.zeros((_L,), jnp.float32)
    ones16 = jnp.full((_L,), 1.0, jnp.float32)
    lanes = lax.iota(jnp.int32, _L)

    def _zero_row(r, carry):
        for c16 in range(_K // _L):
            buf[r, pl.ds(c16 * _L, _L)] = zeros16
        return carry

    lax.fori_loop(0, _CH, _zero_row, 0)

    for g in range(_NCH):
        for gp in range(_CH // _L):
            off = g * _CH + gp * _L
            cols = idx_v[pl.ds(off, _L)]
            rows = lanes + (gp * _L)
            plsc.store_scatter(buf, [rows, cols], ones16)
        pltpu.sync_copy(buf, enc_hbm.at[pl.ds(base + g * _CH, _CH)])
        for gp in range(_CH // _L):
            off = g * _CH + gp * _L
            cols = idx_v[pl.ds(off, _L)]
            rows = lanes + (gp * _L)
            plsc.store_scatter(buf, [rows, cols], zeros16)


def kernel(inputs, W):
    xt = jnp.transpose(inputs, (0, 2, 1))
    wt = W.T
    qt, loss, perp, idx3 = pl.pallas_call(
        _tc_body,
        grid=(_B,),
        in_specs=[
            pl.BlockSpec((1, _D, _S), lambda i: (i, 0, 0)),
            pl.BlockSpec((_D, _K), lambda i: (0, 0)),
        ],
        out_specs=[
            pl.BlockSpec((1, _D, _S), lambda i: (i, 0, 0)),
            pl.BlockSpec(memory_space=pltpu.SMEM),
            pl.BlockSpec(memory_space=pltpu.SMEM),
            pl.BlockSpec((1, 1, _S), lambda i: (i, 0, 0)),
        ],
        out_shape=[
            jax.ShapeDtypeStruct((_B, _D, _S), jnp.float32),
            jax.ShapeDtypeStruct((1, 1), jnp.float32),
            jax.ShapeDtypeStruct((1, 1), jnp.float32),
            jax.ShapeDtypeStruct((_B, 1, _S), jnp.int32),
        ],
        scratch_shapes=[
            pltpu.VMEM((1, _K), jnp.float32),
            pltpu.SMEM((1,), jnp.float32),
            pltpu.VMEM((1, _K), jnp.float32),
            pltpu.VMEM((_S, _K), jnp.float32),
        ],
    )(xt, wt)
    enc = _sc_onehot(idx3.reshape(_N))
    q = jnp.transpose(qt, (0, 2, 1))
    return (loss[0, 0], q, perp[0, 0], enc)


# speedup vs baseline: 2.0067x; 2.0067x over previous
import jax
import jax.numpy as jnp
from jax import lax
from jax.experimental import pallas as pl
from jax.experimental.pallas import tpu as pltpu

_N = 9216
_B = 16
_S = 576
_D = 64
_K = 1024
_COMMIT = 0.25


def _body(xt_ref, wt_ref, enc_ref, qt_ref, loss_ref, perp_ref,
          counts_ref, sse_ref, w2_ref, iota_ref):
    i = pl.program_id(0)
    xt = xt_ref[...].reshape(_D, _S)
    wt = wt_ref[...]

    @pl.when(i == 0)
    def _():
        w2_ref[...] = jnp.sum(wt * wt, axis=0, keepdims=True)
        iota_ref[...] = lax.broadcasted_iota(
            jnp.int32, (_S, _K), 1).astype(jnp.float32)

    x2 = jnp.sum(xt * xt, axis=0, keepdims=True)
    x2c = x2.reshape(_S, 1)
    m2 = lax.dot_general(xt * (-2.0), wt, (((0,), (0,)), ((), ())),
                         preferred_element_type=jnp.float32)
    dist = (x2c + w2_ref[...]) + m2

    dmin = jnp.min(dist, axis=1, keepdims=True)
    iota = iota_ref[...]
    idx = jnp.min(jnp.where(dist == dmin, iota, float(_K)), axis=1,
                  keepdims=True)

    onehot = (iota == idx).astype(jnp.float32)
    enc_ref[...] = onehot

    qt = lax.dot_general(wt, onehot, (((1,), (1,)), ((), ())),
                         preferred_element_type=jnp.float32)
    qt_ref[...] = (xt + (qt - xt)).reshape(1, _D, _S)

    part = jnp.sum((qt - xt) * (qt - xt))
    csum = jnp.sum(onehot, axis=0, keepdims=True)

    @pl.when(i == 0)
    def _():
        sse_ref[0] = part
        counts_ref[...] = csum

    @pl.when(i > 0)
    def _():
        sse_ref[0] += part
        counts_ref[...] += csum

    @pl.when(i == _B - 1)
    def _():
        mse = sse_ref[0] / float(_N * _D)
        loss_ref[0, 0] = mse + _COMMIT * mse
        avg = counts_ref[...] / float(_N)
        ent = jnp.sum(avg * jnp.log(avg + 1e-10))
        perp_ref[0, 0] = jnp.exp(-ent)


def kernel(inputs, W):
    xt = jnp.transpose(inputs, (0, 2, 1))
    wt = W.T
    enc, qt, loss, perp = pl.pallas_call(
        _body,
        grid=(_B,),
        in_specs=[
            pl.BlockSpec((1, _D, _S), lambda i: (i, 0, 0)),
            pl.BlockSpec((_D, _K), lambda i: (0, 0)),
        ],
        out_specs=[
            pl.BlockSpec((_S, _K), lambda i: (i, 0)),
            pl.BlockSpec((1, _D, _S), lambda i: (i, 0, 0)),
            pl.BlockSpec(memory_space=pltpu.SMEM),
            pl.BlockSpec(memory_space=pltpu.SMEM),
        ],
        out_shape=[
            jax.ShapeDtypeStruct((_N, _K), jnp.float32),
            jax.ShapeDtypeStruct((_B, _D, _S), jnp.float32),
            jax.ShapeDtypeStruct((1, 1), jnp.float32),
            jax.ShapeDtypeStruct((1, 1), jnp.float32),
        ],
        scratch_shapes=[
            pltpu.VMEM((1, _K), jnp.float32),
            pltpu.SMEM((1,), jnp.float32),
            pltpu.VMEM((1, _K), jnp.float32),
            pltpu.VMEM((_S, _K), jnp.float32),
        ],
    )(xt, wt)
    q = jnp.transpose(qt, (0, 2, 1))
    return (loss[0, 0], q, perp[0, 0], enc)
